# baseline (speedup 1.0000x reference)
import functools

import jax
import jax.numpy as jnp
from jax import lax
from jax.experimental import pallas as pl
from jax.experimental.pallas import tpu as pltpu
from jax.experimental.pallas import tpu_sc as plsc

N_USERS = 1000000
N_ITEMS = 1000000
N_FACTORS = 64
BATCH = 16384
Y_SCALE = 5.0

NC = 2
NS = 16
NW = NC * NS
B_PER_W = BATCH // NW
IDX_CHUNK = 128
N_CHUNKS = B_PER_W // IDX_CHUNK
L = 16
STRIDE = B_PER_W + 1
HALF = 512000
TR_COLS = 512
TR_GRID = HALF // TR_COLS
WROWS = 256
N_WAVES = B_PER_W // WROWS


def _tr_body(lo_ref, hi_ref, out_ref):
    lo = jnp.transpose(lo_ref[...])
    hi = jnp.transpose(hi_ref[...])
    out_ref[...] = jnp.concatenate([lo, hi], axis=1)


def _transpose_table(wt):
    return pl.pallas_call(
        _tr_body,
        grid=(TR_GRID,),
        in_specs=[
            pl.BlockSpec((N_FACTORS, TR_COLS), lambda c: (0, c)),
            pl.BlockSpec((N_FACTORS, TR_COLS),
                         lambda c: (0, jnp.minimum(
                             c + HALF // TR_COLS,
                             (N_USERS + TR_COLS - 1) // TR_COLS - 1))),
        ],
        out_specs=pl.BlockSpec((TR_COLS, 2 * N_FACTORS), lambda c: (c, 0)),
        out_shape=jax.ShapeDtypeStruct((HALF, 2 * N_FACTORS), jnp.float32),
    )(wt, wt)


def _dot_body(users_hbm, items_hbm, uw2_hbm, iw2_hbm, dots_hbm,
              uidx_v, iidx_v, urow_v, irow_v, urows_v, irows_v,
              stage_v, dots_v, sem):
    wid = lax.axis_index("s") * NC + lax.axis_index("c")
    base = wid * B_PER_W

    for c in range(N_CHUNKS):
        pltpu.sync_copy(users_hbm.at[pl.ds(base + c * IDX_CHUNK, IDX_CHUNK)],
                        uidx_v.at[c])
        pltpu.sync_copy(items_hbm.at[pl.ds(base + c * IDX_CHUNK, IDX_CHUNK)],
                        iidx_v.at[c])

    def adj_body(g, carry):
        c, o = g // (IDX_CHUNK // L), (g % (IDX_CHUNK // L)) * L
        uv = uidx_v[c, pl.ds(o, L)]
        iv = iidx_v[c, pl.ds(o, L)]
        urow_v[c, pl.ds(o, L)] = uv - jnp.where(uv >= HALF, HALF, 0)
        irow_v[c, pl.ds(o, L)] = iv - jnp.where(iv >= HALF, HALF, 0)
        return carry

    lax.fori_loop(0, B_PER_W // L, adj_body, 0)

    lane = lax.iota(jnp.int32, L)
    lane_base = lane * STRIDE

    for w in range(N_WAVES):
        copies = []
        for k in range(WROWS // IDX_CHUNK):
            c = w * (WROWS // IDX_CHUNK) + k
            sl = pl.ds(k * IDX_CHUNK, IDX_CHUNK)
            copies.append(pltpu.async_copy(uw2_hbm.at[urow_v.at[c]],
                                           urows_v.at[sl], sem))
            copies.append(pltpu.async_copy(iw2_hbm.at[irow_v.at[c]],
                                           irows_v.at[sl], sem))
        for cp in copies:
            cp.wait()

        def row_body(g, carry):
            b0 = w * WROWS + g * L
            c, o = b0 // IDX_CHUNK, b0 % IDX_CHUNK
            uoff = jnp.where(uidx_v[c, pl.ds(o, L)] >= HALF, N_FACTORS, 0)
            ioff = jnp.where(iidx_v[c, pl.ds(o, L)] >= HALF, N_FACTORS, 0)
            for j in range(L):
                r = g * L + j
                uo = uoff[j]
                io = ioff[j]
                p = (urows_v[r, pl.ds(uo, L)] * irows_v[r, pl.ds(io, L)]
                     + urows_v[r, pl.ds(uo + L, L)]
                     * irows_v[r, pl.ds(io + L, L)]
                     + urows_v[r, pl.ds(uo + 2 * L, L)]
                     * irows_v[r, pl.ds(io + 2 * L, L)]
                     + urows_v[r, pl.ds(uo + 3 * L, L)]
                     * irows_v[r, pl.ds(io + 3 * L, L)])
                plsc.store_scatter(stage_v, [lane_base + w * WROWS + r], p)
            return carry

        lax.fori_loop(0, WROWS // L, row_body, 0)

    def sum_body(g, carry):
        acc = stage_v[pl.ds(g * L, L)]
        for l in range(1, L):
            acc = acc + stage_v[pl.ds(l * STRIDE + g * L, L)]
        dots_v[pl.ds(g * L, L)] = acc
        return carry

    lax.fori_loop(0, B_PER_W // L, sum_body, 0)

    pltpu.sync_copy(dots_v, dots_hbm.at[pl.ds(base, B_PER_W)])


def _bias_body(users_hbm, items_hbm, ub_hbm, ib_hbm, dots_hbm, out_hbm,
               uidx_v, iidx_v, ub_v, ib_v, dots_v, out_v, sem):
    wid = lax.axis_index("s") * NC + lax.axis_index("c")
    base = wid * B_PER_W

    for c in range(N_CHUNKS):
        pltpu.sync_copy(users_hbm.at[pl.ds(base + c * IDX_CHUNK, IDX_CHUNK)],
                        uidx_v.at[c])
        pltpu.sync_copy(items_hbm.at[pl.ds(base + c * IDX_CHUNK, IDX_CHUNK)],
                        iidx_v.at[c])
    pltpu.sync_copy(dots_hbm.at[pl.ds(base, B_PER_W)], dots_v)

    copies = []
    for c in range(N_CHUNKS):
        sl = pl.ds(c * IDX_CHUNK, IDX_CHUNK)
        copies.append(pltpu.async_copy(ub_hbm.at[uidx_v.at[c]],
                                       ub_v.at[sl], sem))
        copies.append(pltpu.async_copy(ib_hbm.at[iidx_v.at[c]],
                                       ib_v.at[sl], sem))
    for cp in copies:
        cp.wait()

    def epi_body(g, carry):
        s = pl.ds(g * L, L)
        x = dots_v[s] + ub_v[s] + ib_v[s]
        out_v[s] = Y_SCALE / (1.0 + jnp.exp(-x))
        return carry

    lax.fori_loop(0, B_PER_W // L, epi_body, 0)

    pltpu.sync_copy(out_v, out_hbm.at[pl.ds(base, B_PER_W)])


@jax.jit
def _run(users, items, u_weight, i_weight, u_bias, i_bias):
    uw2 = _transpose_table(u_weight.T)
    iw2 = _transpose_table(i_weight.T)

    mesh = plsc.VectorSubcoreMesh(core_axis_name="c", subcore_axis_name="s",
                                  num_cores=NC, num_subcores=NS)
    dot_f = pl.kernel(
        _dot_body,
        out_type=jax.ShapeDtypeStruct((BATCH,), jnp.float32),
        mesh=mesh,
        compiler_params=pltpu.CompilerParams(needs_layout_passes=False,
                                             use_tc_tiling_on_sc=False),
        scratch_types=[
            pltpu.VMEM((N_CHUNKS, IDX_CHUNK), jnp.int32),
            pltpu.VMEM((N_CHUNKS, IDX_CHUNK), jnp.int32),
            pltpu.VMEM((N_CHUNKS, IDX_CHUNK), jnp.int32),
            pltpu.VMEM((N_CHUNKS, IDX_CHUNK), jnp.int32),
            pltpu.VMEM((WROWS, 2 * N_FACTORS), jnp.float32),
            pltpu.VMEM((WROWS, 2 * N_FACTORS), jnp.float32),
            pltpu.VMEM((L * STRIDE,), jnp.float32),
            pltpu.VMEM((B_PER_W,), jnp.float32),
            pltpu.SemaphoreType.DMA,
        ],
    )
    dots = dot_f(users, items, uw2, iw2)

    bias_f = pl.kernel(
        _bias_body,
        out_type=jax.ShapeDtypeStruct((BATCH,), jnp.float32),
        mesh=mesh,
        compiler_params=pltpu.CompilerParams(needs_layout_passes=False,
                                             use_tc_tiling_on_sc=False),
        scratch_types=[
            pltpu.VMEM((N_CHUNKS, IDX_CHUNK), jnp.int32),
            pltpu.VMEM((N_CHUNKS, IDX_CHUNK), jnp.int32),
            pltpu.VMEM((B_PER_W,), jnp.float32),
            pltpu.VMEM((B_PER_W,), jnp.float32),
            pltpu.VMEM((B_PER_W,), jnp.float32),
            pltpu.VMEM((B_PER_W,), jnp.float32),
            pltpu.SemaphoreType.DMA,
        ],
    )
    return bias_f(users, items, u_bias.reshape(N_USERS),
                  i_bias.reshape(N_ITEMS), dots)


def kernel(users, items, u_weight, i_weight, u_bias, i_bias):
    return _run(users.astype(jnp.int32), items.astype(jnp.int32),
                u_weight, i_weight, u_bias, i_bias)


# speedup vs baseline: 1.9803x; 1.9803x over previous
import functools

import jax
import jax.numpy as jnp
from jax import lax
from jax.experimental import pallas as pl
from jax.experimental.pallas import tpu as pltpu
from jax.experimental.pallas import tpu_sc as plsc

N_USERS = 1000000
N_ITEMS = 1000000
N_FACTORS = 64
BATCH = 16384
Y_SCALE = 5.0

NC = 2
NS = 16
NW = NC * NS
B_PER_W = BATCH // NW
IDX_CHUNK = 128
N_CHUNKS = B_PER_W // IDX_CHUNK
L = 16
STRIDE = B_PER_W + 1
HALF = 512000
TR_COLS = 2048
TR_GRID = HALF // TR_COLS
WROWS = 256
N_WAVES = B_PER_W // WROWS


def _tr_body(lo_ref, hi_ref, out_ref):
    lo = jnp.transpose(lo_ref[...])
    hi = jnp.transpose(hi_ref[...])
    out_ref[...] = jnp.concatenate([lo, hi], axis=1)


def _transpose_table(wt):
    return pl.pallas_call(
        _tr_body,
        grid=(TR_GRID,),
        in_specs=[
            pl.BlockSpec((N_FACTORS, TR_COLS), lambda c: (0, c)),
            pl.BlockSpec((N_FACTORS, TR_COLS),
                         lambda c: (0, jnp.minimum(
                             c + HALF // TR_COLS,
                             (N_USERS + TR_COLS - 1) // TR_COLS - 1))),
        ],
        out_specs=pl.BlockSpec((TR_COLS, 2 * N_FACTORS), lambda c: (c, 0)),
        out_shape=jax.ShapeDtypeStruct((HALF, 2 * N_FACTORS), jnp.float32),
        compiler_params=pltpu.CompilerParams(
            dimension_semantics=("arbitrary",)),
    )(wt, wt)


def _dot_body(users_hbm, items_hbm, uw2_hbm, iw2_hbm, dots_hbm,
              uidx_v, iidx_v, urow_v, irow_v, urows_v, irows_v,
              stage_v, dots_v, sem):
    wid = lax.axis_index("s") * NC + lax.axis_index("c")
    base = wid * B_PER_W

    for c in range(N_CHUNKS):
        pltpu.sync_copy(users_hbm.at[pl.ds(base + c * IDX_CHUNK, IDX_CHUNK)],
                        uidx_v.at[c])
        pltpu.sync_copy(items_hbm.at[pl.ds(base + c * IDX_CHUNK, IDX_CHUNK)],
                        iidx_v.at[c])

    def adj_body(g, carry):
        c, o = g // (IDX_CHUNK // L), (g % (IDX_CHUNK // L)) * L
        uv = uidx_v[c, pl.ds(o, L)]
        iv = iidx_v[c, pl.ds(o, L)]
        urow_v[c, pl.ds(o, L)] = uv - jnp.where(uv >= HALF, HALF, 0)
        irow_v[c, pl.ds(o, L)] = iv - jnp.where(iv >= HALF, HALF, 0)
        return carry

    lax.fori_loop(0, B_PER_W // L, adj_body, 0)

    lane = lax.iota(jnp.int32, L)
    lane_base = lane * STRIDE

    for w in range(N_WAVES):
        copies = []
        for k in range(WROWS // IDX_CHUNK):
            c = w * (WROWS // IDX_CHUNK) + k
            sl = pl.ds(k * IDX_CHUNK, IDX_CHUNK)
            copies.append(pltpu.async_copy(uw2_hbm.at[urow_v.at[c]],
                                           urows_v.at[sl], sem))
            copies.append(pltpu.async_copy(iw2_hbm.at[irow_v.at[c]],
                                           irows_v.at[sl], sem))
        for cp in copies:
            cp.wait()

        def row_body(g, carry):
            b0 = w * WROWS + g * L
            c, o = b0 // IDX_CHUNK, b0 % IDX_CHUNK
            uoff = jnp.where(uidx_v[c, pl.ds(o, L)] >= HALF, N_FACTORS, 0)
            ioff = jnp.where(iidx_v[c, pl.ds(o, L)] >= HALF, N_FACTORS, 0)
            for j in range(L):
                r = g * L + j
                uo = uoff[j]
                io = ioff[j]
                p = (urows_v[r, pl.ds(uo, L)] * irows_v[r, pl.ds(io, L)]
                     + urows_v[r, pl.ds(uo + L, L)]
                     * irows_v[r, pl.ds(io + L, L)]
                     + urows_v[r, pl.ds(uo + 2 * L, L)]
                     * irows_v[r, pl.ds(io + 2 * L, L)]
                     + urows_v[r, pl.ds(uo + 3 * L, L)]
                     * irows_v[r, pl.ds(io + 3 * L, L)])
                plsc.store_scatter(stage_v, [lane_base + w * WROWS + r], p)
            return carry

        lax.fori_loop(0, WROWS // L, row_body, 0)

    def sum_body(g, carry):
        acc = stage_v[pl.ds(g * L, L)]
        for l in range(1, L):
            acc = acc + stage_v[pl.ds(l * STRIDE + g * L, L)]
        dots_v[pl.ds(g * L, L)] = acc
        return carry

    lax.fori_loop(0, B_PER_W // L, sum_body, 0)

    pltpu.sync_copy(dots_v, dots_hbm.at[pl.ds(base, B_PER_W)])


def _bias_body(users_hbm, items_hbm, ub_hbm, ib_hbm, dots_hbm, out_hbm,
               uidx_v, iidx_v, ub_v, ib_v, dots_v, out_v, sem):
    wid = lax.axis_index("s") * NC + lax.axis_index("c")
    base = wid * B_PER_W

    for c in range(N_CHUNKS):
        pltpu.sync_copy(users_hbm.at[pl.ds(base + c * IDX_CHUNK, IDX_CHUNK)],
                        uidx_v.at[c])
        pltpu.sync_copy(items_hbm.at[pl.ds(base + c * IDX_CHUNK, IDX_CHUNK)],
                        iidx_v.at[c])
    pltpu.sync_copy(dots_hbm.at[pl.ds(base, B_PER_W)], dots_v)

    copies = []
    for c in range(N_CHUNKS):
        sl = pl.ds(c * IDX_CHUNK, IDX_CHUNK)
        copies.append(pltpu.async_copy(ub_hbm.at[uidx_v.at[c]],
                                       ub_v.at[sl], sem))
        copies.append(pltpu.async_copy(ib_hbm.at[iidx_v.at[c]],
                                       ib_v.at[sl], sem))
    for cp in copies:
        cp.wait()

    def epi_body(g, carry):
        s = pl.ds(g * L, L)
        x = dots_v[s] + ub_v[s] + ib_v[s]
        out_v[s] = Y_SCALE / (1.0 + jnp.exp(-x))
        return carry

    lax.fori_loop(0, B_PER_W // L, epi_body, 0)

    pltpu.sync_copy(out_v, out_hbm.at[pl.ds(base, B_PER_W)])


@jax.jit
def _run(users, items, u_weight, i_weight, u_bias, i_bias):
    uw2 = _transpose_table(u_weight.T)
    iw2 = _transpose_table(i_weight.T)

    mesh = plsc.VectorSubcoreMesh(core_axis_name="c", subcore_axis_name="s",
                                  num_cores=NC, num_subcores=NS)
    dot_f = pl.kernel(
        _dot_body,
        out_type=jax.ShapeDtypeStruct((BATCH,), jnp.float32),
        mesh=mesh,
        compiler_params=pltpu.CompilerParams(needs_layout_passes=False,
                                             use_tc_tiling_on_sc=False),
        scratch_types=[
            pltpu.VMEM((N_CHUNKS, IDX_CHUNK), jnp.int32),
            pltpu.VMEM((N_CHUNKS, IDX_CHUNK), jnp.int32),
            pltpu.VMEM((N_CHUNKS, IDX_CHUNK), jnp.int32),
            pltpu.VMEM((N_CHUNKS, IDX_CHUNK), jnp.int32),
            pltpu.VMEM((WROWS, 2 * N_FACTORS), jnp.float32),
            pltpu.VMEM((WROWS, 2 * N_FACTORS), jnp.float32),
            pltpu.VMEM((L * STRIDE,), jnp.float32),
            pltpu.VMEM((B_PER_W,), jnp.float32),
            pltpu.SemaphoreType.DMA,
        ],
    )
    dots = dot_f(users, items, uw2, iw2)

    bias_f = pl.kernel(
        _bias_body,
        out_type=jax.ShapeDtypeStruct((BATCH,), jnp.float32),
        mesh=mesh,
        compiler_params=pltpu.CompilerParams(needs_layout_passes=False,
                                             use_tc_tiling_on_sc=False),
        scratch_types=[
            pltpu.VMEM((N_CHUNKS, IDX_CHUNK), jnp.int32),
            pltpu.VMEM((N_CHUNKS, IDX_CHUNK), jnp.int32),
            pltpu.VMEM((B_PER_W,), jnp.float32),
            pltpu.VMEM((B_PER_W,), jnp.float32),
            pltpu.VMEM((B_PER_W,), jnp.float32),
            pltpu.VMEM((B_PER_W,), jnp.float32),
            pltpu.SemaphoreType.DMA,
        ],
    )
    return bias_f(users, items, u_bias.reshape(N_USERS),
                  i_bias.reshape(N_ITEMS), dots)


def kernel(users, items, u_weight, i_weight, u_bias, i_bias):
    return _run(users.astype(jnp.int32), items.astype(jnp.int32),
                u_weight, i_weight, u_bias, i_bias)


# speedup vs baseline: 2.4635x; 1.2440x over previous
import functools

import jax
import jax.numpy as jnp
from jax import lax
from jax.experimental import pallas as pl
from jax.experimental.pallas import tpu as pltpu
from jax.experimental.pallas import tpu_sc as plsc

N_USERS = 1000000
N_ITEMS = 1000000
N_FACTORS = 64
BATCH = 16384
Y_SCALE = 5.0

NC = 2
NS = 16
NW = NC * NS
B_PER_W = BATCH // NW
IDX_CHUNK = 128
N_CHUNKS = B_PER_W // IDX_CHUNK
L = 16
STRIDE = B_PER_W + 1
HALF = 512000
TR_COLS = 4096
TR_GRID = HALF // TR_COLS
WROWS = 256
N_WAVES = B_PER_W // WROWS


def _tr_body(lo_ref, hi_ref, out_ref):
    lo = jnp.transpose(lo_ref[...])
    hi = jnp.transpose(hi_ref[...])
    out_ref[...] = jnp.concatenate([lo, hi], axis=1)


def _transpose_table(wt):
    return pl.pallas_call(
        _tr_body,
        grid=(TR_GRID,),
        in_specs=[
            pl.BlockSpec((N_FACTORS, TR_COLS), lambda c: (0, c)),
            pl.BlockSpec((N_FACTORS, TR_COLS),
                         lambda c: (0, jnp.minimum(
                             c + HALF // TR_COLS,
                             (N_USERS + TR_COLS - 1) // TR_COLS - 1))),
        ],
        out_specs=pl.BlockSpec((TR_COLS, 2 * N_FACTORS), lambda c: (c, 0)),
        out_shape=jax.ShapeDtypeStruct((HALF, 2 * N_FACTORS), jnp.float32),
        compiler_params=pltpu.CompilerParams(
            dimension_semantics=("arbitrary",)),
    )(wt, wt)


def _dot_body(users_hbm, items_hbm, uw2_hbm, iw2_hbm, dots_hbm,
              uidx_v, iidx_v, urow_v, irow_v, urows_v, irows_v,
              stage_v, dots_v, sem):
    wid = lax.axis_index("s") * NC + lax.axis_index("c")
    base = wid * B_PER_W

    for c in range(N_CHUNKS):
        pltpu.sync_copy(users_hbm.at[pl.ds(base + c * IDX_CHUNK, IDX_CHUNK)],
                        uidx_v.at[c])
        pltpu.sync_copy(items_hbm.at[pl.ds(base + c * IDX_CHUNK, IDX_CHUNK)],
                        iidx_v.at[c])

    def adj_body(g, carry):
        c, o = g // (IDX_CHUNK // L), (g % (IDX_CHUNK // L)) * L
        uv = uidx_v[c, pl.ds(o, L)]
        iv = iidx_v[c, pl.ds(o, L)]
        urow_v[c, pl.ds(o, L)] = uv - jnp.where(uv >= HALF, HALF, 0)
        irow_v[c, pl.ds(o, L)] = iv - jnp.where(iv >= HALF, HALF, 0)
        return carry

    lax.fori_loop(0, B_PER_W // L, adj_body, 0)

    lane = lax.iota(jnp.int32, L)
    lane_base = lane * STRIDE

    for w in range(N_WAVES):
        copies = []
        for k in range(WROWS // IDX_CHUNK):
            c = w * (WROWS // IDX_CHUNK) + k
            sl = pl.ds(k * IDX_CHUNK, IDX_CHUNK)
            copies.append(pltpu.async_copy(uw2_hbm.at[urow_v.at[c]],
                                           urows_v.at[sl], sem))
            copies.append(pltpu.async_copy(iw2_hbm.at[irow_v.at[c]],
                                           irows_v.at[sl], sem))
        for cp in copies:
            cp.wait()

        def row_body(g, carry):
            b0 = w * WROWS + g * L
            c, o = b0 // IDX_CHUNK, b0 % IDX_CHUNK
            uoff = jnp.where(uidx_v[c, pl.ds(o, L)] >= HALF, N_FACTORS, 0)
            ioff = jnp.where(iidx_v[c, pl.ds(o, L)] >= HALF, N_FACTORS, 0)
            for j in range(L):
                r = g * L + j
                uo = uoff[j]
                io = ioff[j]
                p = (urows_v[r, pl.ds(uo, L)] * irows_v[r, pl.ds(io, L)]
                     + urows_v[r, pl.ds(uo + L, L)]
                     * irows_v[r, pl.ds(io + L, L)]
                     + urows_v[r, pl.ds(uo + 2 * L, L)]
                     * irows_v[r, pl.ds(io + 2 * L, L)]
                     + urows_v[r, pl.ds(uo + 3 * L, L)]
                     * irows_v[r, pl.ds(io + 3 * L, L)])
                plsc.store_scatter(stage_v, [lane_base + w * WROWS + r], p)
            return carry

        lax.fori_loop(0, WROWS // L, row_body, 0)

    def sum_body(g, carry):
        acc = stage_v[pl.ds(g * L, L)]
        for l in range(1, L):
            acc = acc + stage_v[pl.ds(l * STRIDE + g * L, L)]
        dots_v[pl.ds(g * L, L)] = acc
        return carry

    lax.fori_loop(0, B_PER_W // L, sum_body, 0)

    pltpu.sync_copy(dots_v, dots_hbm.at[pl.ds(base, B_PER_W)])


def _bias_body(users_hbm, items_hbm, ub_hbm, ib_hbm, dots_hbm, out_hbm,
               uidx_v, iidx_v, ub_v, ib_v, dots_v, out_v, sem):
    wid = lax.axis_index("s") * NC + lax.axis_index("c")
    base = wid * B_PER_W

    for c in range(N_CHUNKS):
        pltpu.sync_copy(users_hbm.at[pl.ds(base + c * IDX_CHUNK, IDX_CHUNK)],
                        uidx_v.at[c])
        pltpu.sync_copy(items_hbm.at[pl.ds(base + c * IDX_CHUNK, IDX_CHUNK)],
                        iidx_v.at[c])
    pltpu.sync_copy(dots_hbm.at[pl.ds(base, B_PER_W)], dots_v)

    ub1 = ub_hbm.at[0]
    ib1 = ib_hbm.at[0]
    copies = []
    for c in range(N_CHUNKS):
        sl = pl.ds(c * IDX_CHUNK, IDX_CHUNK)
        copies.append(pltpu.async_copy(ub1.at[uidx_v.at[c]],
                                       ub_v.at[sl], sem))
        copies.append(pltpu.async_copy(ib1.at[iidx_v.at[c]],
                                       ib_v.at[sl], sem))
    for cp in copies:
        cp.wait()

    def epi_body(g, carry):
        s = pl.ds(g * L, L)
        x = dots_v[s] + ub_v[s] + ib_v[s]
        out_v[s] = Y_SCALE / (1.0 + jnp.exp(-x))
        return carry

    lax.fori_loop(0, B_PER_W // L, epi_body, 0)

    pltpu.sync_copy(out_v, out_hbm.at[pl.ds(base, B_PER_W)])


@jax.jit
def _run(users, items, u_weight, i_weight, u_bias, i_bias):
    uw2 = _transpose_table(u_weight.T)
    iw2 = _transpose_table(i_weight.T)

    mesh = plsc.VectorSubcoreMesh(core_axis_name="c", subcore_axis_name="s",
                                  num_cores=NC, num_subcores=NS)
    dot_f = pl.kernel(
        _dot_body,
        out_type=jax.ShapeDtypeStruct((BATCH,), jnp.float32),
        mesh=mesh,
        compiler_params=pltpu.CompilerParams(needs_layout_passes=False,
                                             use_tc_tiling_on_sc=False),
        scratch_types=[
            pltpu.VMEM((N_CHUNKS, IDX_CHUNK), jnp.int32),
            pltpu.VMEM((N_CHUNKS, IDX_CHUNK), jnp.int32),
            pltpu.VMEM((N_CHUNKS, IDX_CHUNK), jnp.int32),
            pltpu.VMEM((N_CHUNKS, IDX_CHUNK), jnp.int32),
            pltpu.VMEM((WROWS, 2 * N_FACTORS), jnp.float32),
            pltpu.VMEM((WROWS, 2 * N_FACTORS), jnp.float32),
            pltpu.VMEM((L * STRIDE,), jnp.float32),
            pltpu.VMEM((B_PER_W,), jnp.float32),
            pltpu.SemaphoreType.DMA,
        ],
    )
    dots = dot_f(users, items, uw2, iw2)

    bias_f = pl.kernel(
        _bias_body,
        out_type=jax.ShapeDtypeStruct((BATCH,), jnp.float32),
        mesh=mesh,
        compiler_params=pltpu.CompilerParams(needs_layout_passes=False,
                                             use_tc_tiling_on_sc=False),
        scratch_types=[
            pltpu.VMEM((N_CHUNKS, IDX_CHUNK), jnp.int32),
            pltpu.VMEM((N_CHUNKS, IDX_CHUNK), jnp.int32),
            pltpu.VMEM((B_PER_W,), jnp.float32),
            pltpu.VMEM((B_PER_W,), jnp.float32),
            pltpu.VMEM((B_PER_W,), jnp.float32),
            pltpu.VMEM((B_PER_W,), jnp.float32),
            pltpu.SemaphoreType.DMA,
        ],
    )
    return bias_f(users, items, u_bias.T, i_bias.T, dots)


def kernel(users, items, u_weight, i_weight, u_bias, i_bias):
    return _run(users.astype(jnp.int32), items.astype(jnp.int32),
                u_weight, i_weight, u_bias, i_bias)


# speedup vs baseline: 2.6145x; 1.0613x over previous
import functools

import jax
import jax.numpy as jnp
from jax import lax
from jax.experimental import pallas as pl
from jax.experimental.pallas import tpu as pltpu
from jax.experimental.pallas import tpu_sc as plsc

N_USERS = 1000000
N_ITEMS = 1000000
N_FACTORS = 64
BATCH = 16384
Y_SCALE = 5.0

NC = 2
NS = 16
NW = NC * NS
B_PER_W = BATCH // NW
IDX_CHUNK = 128
N_CHUNKS = B_PER_W // IDX_CHUNK
L = 16
STRIDE = B_PER_W + 1
HALF = 512000
TR_COLS = 6400
TR_GRID = HALF // TR_COLS
WROWS = 256
N_WAVES = B_PER_W // WROWS


def _tr_body(lo_ref, hi_ref, out_ref):
    lo = jnp.transpose(lo_ref[...])
    hi = jnp.transpose(hi_ref[...])
    out_ref[...] = jnp.concatenate([lo, hi], axis=1)


def _transpose_table(wt):
    return pl.pallas_call(
        _tr_body,
        grid=(TR_GRID,),
        in_specs=[
            pl.BlockSpec((N_FACTORS, TR_COLS), lambda c: (0, c)),
            pl.BlockSpec((N_FACTORS, TR_COLS),
                         lambda c: (0, jnp.minimum(
                             c + HALF // TR_COLS,
                             (N_USERS + TR_COLS - 1) // TR_COLS - 1))),
        ],
        out_specs=pl.BlockSpec((TR_COLS, 2 * N_FACTORS), lambda c: (c, 0)),
        out_shape=jax.ShapeDtypeStruct((HALF, 2 * N_FACTORS), jnp.float32),
        compiler_params=pltpu.CompilerParams(
            dimension_semantics=("arbitrary",)),
    )(wt, wt)


def _dot_body(users_hbm, items_hbm, uw2_hbm, iw2_hbm, bsum_hbm, out_hbm,
              uidx_v, iidx_v, urow_v, irow_v, urows_v, irows_v,
              stage_v, bsum_v, out_v, sem):
    wid = lax.axis_index("s") * NC + lax.axis_index("c")
    base = wid * B_PER_W

    for c in range(N_CHUNKS):
        pltpu.sync_copy(users_hbm.at[pl.ds(base + c * IDX_CHUNK, IDX_CHUNK)],
                        uidx_v.at[c])
        pltpu.sync_copy(items_hbm.at[pl.ds(base + c * IDX_CHUNK, IDX_CHUNK)],
                        iidx_v.at[c])
    pltpu.sync_copy(bsum_hbm.at[pl.ds(base, B_PER_W)], bsum_v)

    def adj_body(g, carry):
        c, o = g // (IDX_CHUNK // L), (g % (IDX_CHUNK // L)) * L
        uv = uidx_v[c, pl.ds(o, L)]
        iv = iidx_v[c, pl.ds(o, L)]
        urow_v[c, pl.ds(o, L)] = uv - jnp.where(uv >= HALF, HALF, 0)
        irow_v[c, pl.ds(o, L)] = iv - jnp.where(iv >= HALF, HALF, 0)
        return carry

    lax.fori_loop(0, B_PER_W // L, adj_body, 0)

    lane = lax.iota(jnp.int32, L)
    lane_base = lane * STRIDE

    for w in range(N_WAVES):
        copies = []
        for k in range(WROWS // IDX_CHUNK):
            c = w * (WROWS // IDX_CHUNK) + k
            sl = pl.ds(k * IDX_CHUNK, IDX_CHUNK)
            copies.append(pltpu.async_copy(uw2_hbm.at[urow_v.at[c]],
                                           urows_v.at[sl], sem))
            copies.append(pltpu.async_copy(iw2_hbm.at[irow_v.at[c]],
                                           irows_v.at[sl], sem))
        for cp in copies:
            cp.wait()

        def row_body(g, carry):
            b0 = w * WROWS + g * L
            c, o = b0 // IDX_CHUNK, b0 % IDX_CHUNK
            uoff = jnp.where(uidx_v[c, pl.ds(o, L)] >= HALF, N_FACTORS, 0)
            ioff = jnp.where(iidx_v[c, pl.ds(o, L)] >= HALF, N_FACTORS, 0)
            for j in range(L):
                r = g * L + j
                uo = uoff[j]
                io = ioff[j]
                p = (urows_v[r, pl.ds(uo, L)] * irows_v[r, pl.ds(io, L)]
                     + urows_v[r, pl.ds(uo + L, L)]
                     * irows_v[r, pl.ds(io + L, L)]
                     + urows_v[r, pl.ds(uo + 2 * L, L)]
                     * irows_v[r, pl.ds(io + 2 * L, L)]
                     + urows_v[r, pl.ds(uo + 3 * L, L)]
                     * irows_v[r, pl.ds(io + 3 * L, L)])
                plsc.store_scatter(stage_v, [lane_base + w * WROWS + r], p)
            return carry

        lax.fori_loop(0, WROWS // L, row_body, 0)

    def sum_body(g, carry):
        s = pl.ds(g * L, L)
        acc = bsum_v[s]
        for l in range(L):
            acc = acc + stage_v[pl.ds(l * STRIDE + g * L, L)]
        out_v[s] = Y_SCALE / (1.0 + jnp.exp(-acc))
        return carry

    lax.fori_loop(0, B_PER_W // L, sum_body, 0)

    pltpu.sync_copy(out_v, out_hbm.at[pl.ds(base, B_PER_W)])


def _bias_body(users_hbm, items_hbm, ub_hbm, ib_hbm, bsum_hbm,
               uidx_v, iidx_v, ub_v, ib_v, bsum_v, sem):
    wid = lax.axis_index("s") * NC + lax.axis_index("c")
    base = wid * B_PER_W

    for c in range(N_CHUNKS):
        pltpu.sync_copy(users_hbm.at[pl.ds(base + c * IDX_CHUNK, IDX_CHUNK)],
                        uidx_v.at[c])
        pltpu.sync_copy(items_hbm.at[pl.ds(base + c * IDX_CHUNK, IDX_CHUNK)],
                        iidx_v.at[c])

    ub1 = ub_hbm.at[0]
    ib1 = ib_hbm.at[0]
    copies = []
    for c in range(N_CHUNKS):
        sl = pl.ds(c * IDX_CHUNK, IDX_CHUNK)
        copies.append(pltpu.async_copy(ub1.at[uidx_v.at[c]],
                                       ub_v.at[sl], sem))
        copies.append(pltpu.async_copy(ib1.at[iidx_v.at[c]],
                                       ib_v.at[sl], sem))
    for cp in copies:
        cp.wait()

    def epi_body(g, carry):
        s = pl.ds(g * L, L)
        bsum_v[s] = ub_v[s] + ib_v[s]
        return carry

    lax.fori_loop(0, B_PER_W // L, epi_body, 0)

    pltpu.sync_copy(bsum_v, bsum_hbm.at[pl.ds(base, B_PER_W)])


@jax.jit
def _run(users, items, u_weight, i_weight, u_bias, i_bias):
    uw2 = _transpose_table(u_weight.T)
    iw2 = _transpose_table(i_weight.T)

    mesh = plsc.VectorSubcoreMesh(core_axis_name="c", subcore_axis_name="s",
                                  num_cores=NC, num_subcores=NS)
    bias_f = pl.kernel(
        _bias_body,
        out_type=jax.ShapeDtypeStruct((BATCH,), jnp.float32),
        mesh=mesh,
        compiler_params=pltpu.CompilerParams(needs_layout_passes=False,
                                             use_tc_tiling_on_sc=False),
        scratch_types=[
            pltpu.VMEM((N_CHUNKS, IDX_CHUNK), jnp.int32),
            pltpu.VMEM((N_CHUNKS, IDX_CHUNK), jnp.int32),
            pltpu.VMEM((B_PER_W,), jnp.float32),
            pltpu.VMEM((B_PER_W,), jnp.float32),
            pltpu.VMEM((B_PER_W,), jnp.float32),
            pltpu.SemaphoreType.DMA,
        ],
    )
    bsum = bias_f(users, items, u_bias.T, i_bias.T)

    dot_f = pl.kernel(
        _dot_body,
        out_type=jax.ShapeDtypeStruct((BATCH,), jnp.float32),
        mesh=mesh,
        compiler_params=pltpu.CompilerParams(needs_layout_passes=False,
                                             use_tc_tiling_on_sc=False),
        scratch_types=[
            pltpu.VMEM((N_CHUNKS, IDX_CHUNK), jnp.int32),
            pltpu.VMEM((N_CHUNKS, IDX_CHUNK), jnp.int32),
            pltpu.VMEM((N_CHUNKS, IDX_CHUNK), jnp.int32),
            pltpu.VMEM((N_CHUNKS, IDX_CHUNK), jnp.int32),
            pltpu.VMEM((WROWS, 2 * N_FACTORS), jnp.float32),
            pltpu.VMEM((WROWS, 2 * N_FACTORS), jnp.float32),
            pltpu.VMEM((L * STRIDE,), jnp.float32),
            pltpu.VMEM((B_PER_W,), jnp.float32),
            pltpu.VMEM((B_PER_W,), jnp.float32),
            pltpu.SemaphoreType.DMA,
        ],
    )
    out = dot_f(users, items, uw2, iw2, bsum)
    return out


def kernel(users, items, u_weight, i_weight, u_bias, i_bias):
    return _run(users.astype(jnp.int32), items.astype(jnp.int32),
                u_weight, i_weight, u_bias, i_bias)
